# baseline (speedup 1.0000x reference)
import functools

import jax
import jax.numpy as jnp
from jax import lax
from jax.experimental import pallas as pl
from jax.experimental.pallas import tpu as pltpu
from jax.experimental.pallas import tpu_sc as plsc

NU, NM, NT = 100000, 50000, 5000
D_FEAT, H = 20, 128
BK = 1024
NUP, NMP, NTP = 100352, 50176, 5120

CH = 2048
GB = 128
R_BIG = 8192
R_TAG = 4096
ACC_ROWS = R_BIG + 128
EPR = 524288
EPT = 131072

_RATES = (NM, R_BIG, 4)
_REV = (NU, R_BIG, 7)
_HT = (NT, R_TAG, 1)
_TO = (NM, R_BIG, 4)


def _run_agg(c, s, h_tbl, src_e, dst_e, epad, R, npass, s_out, d_out, zer,
             acc, dacc, src_ch, dst_ch, cbs, cbd, fsrc, fdst, rows,
             cpb1, z1, ones_v, sem):
    nchunks = epad // 16 // CH
    tile_base = s * (epad // 16)
    nzb = (R + 128) // 128
    ncp = R // 128
    do_deg = d_out is not None

    def flush():
        for j in range(8):
            fsrc[pl.ds(j * 16, 16)] = cbs[pl.ds(j * 16, 16)]
            fdst[pl.ds(j * 16, 16)] = cbd[pl.ds(j * 16, 16)]
        pltpu.async_copy(h_tbl.at[fsrc], rows, sem).wait()
        pltpu.sync_copy(rows, acc.at[fdst], add=True)
        if do_deg:
            pltpu.sync_copy(ones_v, dacc.at[fdst], add=True)

    def pass_body(p, _):
        lo = p * (2 * R) + c * R
        pltpu.sync_copy(zer, rows)

        def zero_blk(j, _):
            blk = s + j * 16

            @pl.when(blk < nzb)
            def _():
                pltpu.sync_copy(rows, acc.at[pl.ds(blk * 128, 128)])
                if do_deg:
                    pltpu.sync_copy(z1, dacc.at[pl.ds(blk * 128, 128)])
            return 0

        lax.fori_loop(0, -(-nzb // 16), zero_blk, 0)
        plsc.subcore_barrier()

        def chunk_body(ci, cnt):
            eb = tile_base + ci * CH
            pltpu.sync_copy(src_e.at[pl.ds(eb, CH)], src_ch)
            pltpu.sync_copy(dst_e.at[pl.ds(eb, CH)], dst_ch)

            def vec_body(v, cnt):
                sv = src_ch[pl.ds(v * 16, 16)]
                dv = dst_ch[pl.ds(v * 16, 16)]
                m = (dv >= lo) & (dv < lo + R)
                mi = m.astype(jnp.int32)
                pos = jnp.maximum(cnt + plsc.cumsum(mi) - 1, 0)
                plsc.store_scatter(cbs, [pos], sv, mask=m)
                plsc.store_scatter(cbd, [pos], dv - lo, mask=m)
                cnt = cnt + jnp.sum(mi)

                @pl.when(cnt >= GB)
                def _():
                    for j in range(8):
                        fsrc[pl.ds(j * 16, 16)] = cbs[pl.ds(j * 16, 16)]
                        fdst[pl.ds(j * 16, 16)] = cbd[pl.ds(j * 16, 16)]
                    for j in range(8):
                        cbs[pl.ds(j * 16, 16)] = cbs[pl.ds(GB + j * 16, 16)]
                        cbd[pl.ds(j * 16, 16)] = cbd[pl.ds(GB + j * 16, 16)]
                    pltpu.async_copy(h_tbl.at[fsrc], rows, sem).wait()
                    pltpu.sync_copy(rows, acc.at[fdst], add=True)
                    if do_deg:
                        pltpu.sync_copy(ones_v, dacc.at[fdst], add=True)

                return jnp.where(cnt >= GB, cnt - GB, cnt)

            return lax.fori_loop(0, CH // 16, vec_body, cnt)

        cnt = lax.fori_loop(0, nchunks, chunk_body, jnp.int32(0))

        iota = lax.iota(jnp.int32, 16)
        for j in range(8):
            lane = iota + j * 16
            sv = cbs[pl.ds(j * 16, 16)]
            dv = cbd[pl.ds(j * 16, 16)]
            keep = lane < cnt
            cbs[pl.ds(j * 16, 16)] = jnp.where(keep, sv, 0)
            cbd[pl.ds(j * 16, 16)] = jnp.where(keep, dv, R)

        @pl.when(cnt > 0)
        def _():
            flush()

        plsc.subcore_barrier()
        def cp_blk(j, _):
            blk = s + j * 16
            pltpu.sync_copy(acc.at[pl.ds(blk * 128, 128)], rows)
            pltpu.sync_copy(rows, s_out.at[pl.ds(lo + blk * 128, 128)])
            if do_deg:
                pltpu.sync_copy(dacc.at[pl.ds(blk * 128, 128)], cpb1)
                pltpu.sync_copy(cpb1, d_out.at[pl.ds(lo + blk * 128, 128)])
            return 0

        lax.fori_loop(0, ncp // 16, cp_blk, 0)
        plsc.subcore_barrier()
        return 0

    lax.fori_loop(0, npass, pass_body, 0)


def _make_agg_kernel(layer0: bool):
    mesh = plsc.VectorSubcoreMesh(core_axis_name="c", subcore_axis_name="s",
                                  num_cores=2, num_subcores=16)

    out_type = [
        jax.ShapeDtypeStruct((2 * R_BIG * _RATES[2], H), jnp.float32),
        jax.ShapeDtypeStruct((2 * R_BIG * _REV[2], H), jnp.float32),
        jax.ShapeDtypeStruct((2 * R_BIG * _TO[2], H), jnp.float32),
    ]
    if layer0:
        out_type.append(jax.ShapeDtypeStruct((2 * R_TAG, H), jnp.float32))
        out_type += [
            jax.ShapeDtypeStruct((2 * R_BIG * _RATES[2],), jnp.float32),
            jax.ShapeDtypeStruct((2 * R_BIG * _REV[2],), jnp.float32),
            jax.ShapeDtypeStruct((2 * R_BIG * _TO[2],), jnp.float32),
            jax.ShapeDtypeStruct((2 * R_TAG,), jnp.float32),
        ]

    scratch = [
        pltpu.VMEM_SHARED((ACC_ROWS, H), jnp.float32),
        pltpu.VMEM_SHARED((ACC_ROWS,), jnp.float32),
        pltpu.VMEM((CH,), jnp.int32),
        pltpu.VMEM((CH,), jnp.int32),
        pltpu.VMEM((256,), jnp.int32),
        pltpu.VMEM((256,), jnp.int32),
        pltpu.VMEM((GB,), jnp.int32),
        pltpu.VMEM((GB,), jnp.int32),
        pltpu.VMEM((GB, H), jnp.float32),
        pltpu.VMEM((GB,), jnp.float32),
        pltpu.VMEM((GB,), jnp.float32),
        pltpu.VMEM((GB,), jnp.float32),
        pltpu.SemaphoreType.DMA,
    ]

    def body(h_user, h_movie, h_tag,
             r_src, r_dst, v_src, v_dst, o_src, o_dst, ht_src, ht_dst,
             ones1h, zer, zer1, *rest):
        if layer0:
            (s_rates, s_rev, s_to, s_ht, d_rates, d_rev, d_to, d_ht,
             acc, dacc, src_ch, dst_ch, cbs, cbd, fsrc, fdst, rows,
             cpb1, z1, ones_v, sem) = rest
        else:
            (s_rates, s_rev, s_to,
             acc, dacc, src_ch, dst_ch, cbs, cbd, fsrc, fdst, rows,
             cpb1, z1, ones_v, sem) = rest
            d_rates = d_rev = d_to = d_ht = s_ht = None
        c = lax.axis_index("c")
        s = lax.axis_index("s")
        pltpu.sync_copy(ones1h, ones_v)
        pltpu.sync_copy(zer1, z1)
        common = dict(zer=zer, acc=acc, dacc=dacc, src_ch=src_ch,
                      dst_ch=dst_ch, cbs=cbs, cbd=cbd, fsrc=fsrc, fdst=fdst,
                      rows=rows, cpb1=cpb1, z1=z1, ones_v=ones_v, sem=sem)
        _run_agg(c, s, h_user, r_src, r_dst, EPR, _RATES[1], _RATES[2],
                 s_rates, d_rates, **common)
        _run_agg(c, s, h_movie, v_src, v_dst, EPR, _REV[1], _REV[2],
                 s_rev, d_rev, **common)
        _run_agg(c, s, h_tag, o_src, o_dst, EPT, _TO[1], _TO[2],
                 s_to, d_to, **common)
        if layer0:
            _run_agg(c, s, h_movie, ht_src, ht_dst, EPT, _HT[1], _HT[2],
                     s_ht, d_ht, **common)

    return pl.kernel(body, out_type=tuple(out_type), mesh=mesh,
                     scratch_types=scratch,
                     compiler_params=pltpu.CompilerParams(
                         needs_layout_passes=False))


_agg_cache = {}


def _agg(layer0, *args):
    if layer0 not in _agg_cache:
        _agg_cache[layer0] = _make_agg_kernel(layer0)
    return _agg_cache[layer0](*args)


_agg_l0 = functools.partial(_agg, True)
_agg_l1 = functools.partial(_agg, False)


def _proj_movie(movie_genre_p, W_proj, b_proj):
    def body(g_ref, w_ref, b_ref, o_ref):
        o_ref[...] = jnp.dot(g_ref[...], w_ref[...],
                             preferred_element_type=jnp.float32) + b_ref[...]

    return pl.pallas_call(
        body,
        grid=(NMP // BK,),
        in_specs=[pl.BlockSpec((BK, D_FEAT), lambda i: (i, 0)),
                  pl.BlockSpec((D_FEAT, H), lambda i: (0, 0)),
                  pl.BlockSpec((1, H), lambda i: (0, 0))],
        out_specs=pl.BlockSpec((BK, H), lambda i: (i, 0)),
        out_shape=jax.ShapeDtypeStruct((NMP, H), jnp.float32),
    )(movie_genre_p, W_proj, b_proj.reshape(1, H))


def _eye128():
    ri = lax.broadcasted_iota(jnp.int32, (H, H), 0)
    ci = lax.broadcasted_iota(jnp.int32, (H, H), 1)
    return (ri == ci).astype(jnp.float32)


def _dcol(eye, d2, b):
    drow = d2[b:b + 1, :]
    return lax.dot_general(eye, drow, (((1,), (1,)), ((), ())),
                           preferred_element_type=jnp.float32)


def _dense1(h, s_n, deg2, Ws, Wn, b):
    n = h.shape[0]

    def body(h_ref, s_ref, d_ref, ws_ref, wn_ref, b_ref, o_ref):
        eye = _eye128()
        x = jnp.dot(h_ref[...], ws_ref[...],
                    preferred_element_type=jnp.float32) + b_ref[...]
        for bb in range(BK // H):
            dcol = _dcol(eye, d_ref[...], bb)
            mb = s_ref[pl.ds(bb * H, H), :] * (1.0 / jnp.maximum(dcol, 1.0))
            zb = x[bb * H:(bb + 1) * H, :] + jnp.dot(
                mb, wn_ref[...], preferred_element_type=jnp.float32)
            o_ref[pl.ds(bb * H, H), :] = jnp.where(zb >= 0, zb, 0.1 * zb)

    row = lambda i: (i, 0)
    full = lambda i: (0, 0)
    return pl.pallas_call(
        body,
        grid=(n // BK,),
        in_specs=[pl.BlockSpec((BK, H), row),
                  pl.BlockSpec((BK, H), row),
                  pl.BlockSpec((BK // H, H), row),
                  pl.BlockSpec((H, H), full),
                  pl.BlockSpec((H, H), full),
                  pl.BlockSpec((1, H), full)],
        out_specs=pl.BlockSpec((BK, H), row),
        out_shape=jax.ShapeDtypeStruct((n, H), jnp.float32),
    )(h, s_n, deg2, Ws, Wn, b.reshape(1, H))


def _dense2(h, s1, d1, s2, d2, Wsa, Wsb, Wn1, Wn2, b):
    n = h.shape[0]

    def body(h_ref, s1_ref, d1_ref, s2_ref, d2_ref,
             wsa_ref, wsb_ref, wn1_ref, wn2_ref, b_ref, o_ref):
        eye = _eye128()
        ws = wsa_ref[...] + wsb_ref[...]
        x = jnp.dot(h_ref[...], ws,
                    preferred_element_type=jnp.float32) + b_ref[...]
        for bb in range(BK // H):
            c1 = _dcol(eye, d1_ref[...], bb)
            c2 = _dcol(eye, d2_ref[...], bb)
            m1 = s1_ref[pl.ds(bb * H, H), :] * (1.0 / jnp.maximum(c1, 1.0))
            m2 = s2_ref[pl.ds(bb * H, H), :] * (1.0 / jnp.maximum(c2, 1.0))
            zb = x[bb * H:(bb + 1) * H, :]
            zb = zb + jnp.dot(m1, wn1_ref[...], preferred_element_type=jnp.float32)
            zb = zb + jnp.dot(m2, wn2_ref[...], preferred_element_type=jnp.float32)
            o_ref[pl.ds(bb * H, H), :] = jnp.where(zb >= 0, zb, 0.1 * zb)

    row = lambda i: (i, 0)
    full = lambda i: (0, 0)
    return pl.pallas_call(
        body,
        grid=(n // BK,),
        in_specs=[pl.BlockSpec((BK, H), row),
                  pl.BlockSpec((BK, H), row),
                  pl.BlockSpec((BK // H, H), row),
                  pl.BlockSpec((BK, H), row),
                  pl.BlockSpec((BK // H, H), row),
                  pl.BlockSpec((H, H), full),
                  pl.BlockSpec((H, H), full),
                  pl.BlockSpec((H, H), full),
                  pl.BlockSpec((H, H), full),
                  pl.BlockSpec((1, H), full)],
        out_specs=pl.BlockSpec((BK, H), row),
        out_shape=jax.ShapeDtypeStruct((n, H), jnp.float32),
    )(h, s1, d1, s2, d2, Wsa, Wsb, Wn1, Wn2, b.reshape(1, H))


def _pad_edges(idx, epad, fill):
    pad = jnp.full((epad - idx.shape[0],), fill, dtype=jnp.int32)
    return jnp.concatenate([idx.astype(jnp.int32), pad])


def _pad_rows(x, n):
    return jnp.pad(x, ((0, n - x.shape[0]), (0, 0)))


def _deg2(d_flat, npadded):
    return d_flat.reshape(-1, H)[: npadded // H]


@jax.jit
def kernel(movie_genre, rates_src, rates_dst, has_tag_src, has_tag_dst,
           user_emb, tag_emb, W_proj, b_proj, W_self, W_neigh, b_conv):
    r_src = _pad_edges(rates_src, EPR, 0)
    r_dst = _pad_edges(rates_dst, EPR, NM)
    v_src = _pad_edges(rates_dst, EPR, 0)
    v_dst = _pad_edges(rates_src, EPR, NU)
    ht_src = _pad_edges(has_tag_src, EPT, 0)
    ht_dst = _pad_edges(has_tag_dst, EPT, NT)
    o_src = _pad_edges(has_tag_dst, EPT, 0)
    o_dst = _pad_edges(has_tag_src, EPT, NM)

    ones1 = jnp.ones((GB,), jnp.float32)
    zer = jnp.zeros((GB, H), jnp.float32)
    zer1 = jnp.zeros((GB,), jnp.float32)

    h_u0 = _pad_rows(user_emb, NUP)
    h_t0 = _pad_rows(tag_emb, NTP)
    h_m0 = _proj_movie(_pad_rows(movie_genre, NMP), W_proj, b_proj)

    (s_rates, s_rev, s_to, s_ht, d_rates, d_rev, d_to, d_ht) = _agg_l0(
        h_u0, h_m0, h_t0,
        r_src, r_dst, v_src, v_dst, o_src, o_dst, ht_src, ht_dst,
        ones1, zer, zer1)

    d2_rates = _deg2(d_rates, NMP)
    d2_rev = _deg2(d_rev, NUP)
    d2_to = _deg2(d_to, NMP)
    d2_ht = _deg2(d_ht, NTP)
    h_u1 = _dense1(h_u0, s_rev[:NUP], d2_rev,
                   W_self[0, 1], W_neigh[0, 1], b_conv[0, 1])
    h_t1 = _dense1(h_t0, s_ht[:NTP], d2_ht,
                   W_self[0, 2], W_neigh[0, 2], b_conv[0, 2])
    h_m1 = _dense2(h_m0, s_rates[:NMP], d2_rates, s_to[:NMP], d2_to,
                   W_self[0, 0], W_self[0, 3], W_neigh[0, 0], W_neigh[0, 3],
                   b_conv[0, 0] + b_conv[0, 3])

    (s_rates1, s_rev1, s_to1) = _agg_l1(
        h_u1, h_m1, h_t1,
        r_src, r_dst, v_src, v_dst, o_src, o_dst, ht_src, ht_dst,
        ones1, zer, zer1)

    h_u2 = _dense1(h_u1, s_rev1[:NUP], d2_rev,
                   W_self[1, 1], W_neigh[1, 1], b_conv[1, 1])
    h_m2 = _dense2(h_m1, s_rates1[:NMP], d2_rates, s_to1[:NMP], d2_to,
                   W_self[1, 0], W_self[1, 3], W_neigh[1, 0], W_neigh[1, 3],
                   b_conv[1, 0] + b_conv[1, 3])
    return (h_u2[:NU], h_m2[:NM])


# speedup vs baseline: 1.0318x; 1.0318x over previous
import functools

import jax
import jax.numpy as jnp
from jax import lax
from jax.experimental import pallas as pl
from jax.experimental.pallas import tpu as pltpu
from jax.experimental.pallas import tpu_sc as plsc

NU, NM, NT = 100000, 50000, 5000
D_FEAT, H = 20, 128
BK = 1024
NUP, NMP, NTP = 100352, 50176, 5120

CH = 2048
GB = 128
R_BIG = 8192
R_TAG = 4096
ACC_ROWS = R_BIG + 128
EPR = 524288
EPT = 131072

_RATES = (NM, R_BIG, 4)
_REV = (NU, R_BIG, 7)
_HT = (NT, R_TAG, 1)
_TO = (NM, R_BIG, 4)


def _run_agg(c, s, h_tbl, src_e, dst_e, epad, R, npass, s_out, d_out, zer,
             acc, dacc, src_ch, dst_ch, cbs, cbd, fsrcA, fdstA, fsrcB,
             fdstB, rowsA, rowsB, cpb1, z1, ones_v, semA, semB):
    nchunks = epad // 16 // CH
    tile_base = s * (epad // 16)
    nzb = (R + 128) // 128
    ncp = R // 128
    do_deg = d_out is not None
    rows = rowsA

    def stage(base, fsrc, fdst):
        for j in range(8):
            fsrc[pl.ds(j * 16, 16)] = cbs[pl.ds(base + j * 16, 16)]
            fdst[pl.ds(j * 16, 16)] = cbd[pl.ds(base + j * 16, 16)]

    def drain(g, fdst, rows_s):
        g.wait()
        pltpu.sync_copy(rows_s, acc.at[fdst], add=True)
        if do_deg:
            pltpu.sync_copy(ones_v, dacc.at[fdst], add=True)

    def flush2():
        stage(0, fsrcA, fdstA)
        stage(GB, fsrcB, fdstB)
        ga = pltpu.async_copy(h_tbl.at[fsrcA], rowsA, semA)
        gb = pltpu.async_copy(h_tbl.at[fsrcB], rowsB, semB)
        drain(ga, fdstA, rowsA)
        drain(gb, fdstB, rowsB)

    def pass_body(p, _):
        lo = p * (2 * R) + c * R
        pltpu.sync_copy(zer, rows)

        def zero_blk(j, _):
            blk = s + j * 16

            @pl.when(blk < nzb)
            def _():
                pltpu.sync_copy(rows, acc.at[pl.ds(blk * 128, 128)])
                if do_deg:
                    pltpu.sync_copy(z1, dacc.at[pl.ds(blk * 128, 128)])
            return 0

        lax.fori_loop(0, -(-nzb // 16), zero_blk, 0)
        plsc.subcore_barrier()

        def chunk_body(ci, cnt):
            eb = tile_base + ci * CH
            pltpu.sync_copy(src_e.at[pl.ds(eb, CH)], src_ch)
            pltpu.sync_copy(dst_e.at[pl.ds(eb, CH)], dst_ch)

            def vec_body(v, cnt):
                sv = src_ch[pl.ds(v * 16, 16)]
                dv = dst_ch[pl.ds(v * 16, 16)]
                m = (dv >= lo) & (dv < lo + R)
                mi = m.astype(jnp.int32)
                pos = jnp.maximum(cnt + plsc.cumsum(mi) - 1, 0)
                plsc.store_scatter(cbs, [pos], sv, mask=m)
                plsc.store_scatter(cbd, [pos], dv - lo, mask=m)
                cnt = cnt + jnp.sum(mi)

                @pl.when(cnt >= 2 * GB)
                def _():
                    flush2()
                    for j in range(16):
                        cbs[pl.ds(j * 16, 16)] = cbs[pl.ds(2 * GB + j * 16, 16)]
                        cbd[pl.ds(j * 16, 16)] = cbd[pl.ds(2 * GB + j * 16, 16)]

                return jnp.where(cnt >= 2 * GB, cnt - 2 * GB, cnt)

            return lax.fori_loop(0, CH // 16, vec_body, cnt)

        cnt = lax.fori_loop(0, nchunks, chunk_body, jnp.int32(0))

        iota = lax.iota(jnp.int32, 16)
        for j in range(16):
            lane = iota + j * 16
            sv = cbs[pl.ds(j * 16, 16)]
            dv = cbd[pl.ds(j * 16, 16)]
            keep = lane < cnt
            cbs[pl.ds(j * 16, 16)] = jnp.where(keep, sv, 0)
            cbd[pl.ds(j * 16, 16)] = jnp.where(keep, dv, R)

        @pl.when(cnt > 0)
        def _():
            stage(0, fsrcA, fdstA)
            ga = pltpu.async_copy(h_tbl.at[fsrcA], rowsA, semA)
            drain(ga, fdstA, rowsA)

        @pl.when(cnt > GB)
        def _():
            stage(GB, fsrcB, fdstB)
            gb = pltpu.async_copy(h_tbl.at[fsrcB], rowsB, semB)
            drain(gb, fdstB, rowsB)

        plsc.subcore_barrier()
        def cp_blk(j, _):
            blk = s + j * 16
            pltpu.sync_copy(acc.at[pl.ds(blk * 128, 128)], rows)
            pltpu.sync_copy(rows, s_out.at[pl.ds(lo + blk * 128, 128)])
            if do_deg:
                pltpu.sync_copy(dacc.at[pl.ds(blk * 128, 128)], cpb1)
                pltpu.sync_copy(cpb1, d_out.at[pl.ds(lo + blk * 128, 128)])
            return 0

        lax.fori_loop(0, ncp // 16, cp_blk, 0)
        plsc.subcore_barrier()
        return 0

    lax.fori_loop(0, npass, pass_body, 0)


def _make_agg_kernel(layer0: bool):
    mesh = plsc.VectorSubcoreMesh(core_axis_name="c", subcore_axis_name="s",
                                  num_cores=2, num_subcores=16)

    out_type = [
        jax.ShapeDtypeStruct((2 * R_BIG * _RATES[2], H), jnp.float32),
        jax.ShapeDtypeStruct((2 * R_BIG * _REV[2], H), jnp.float32),
        jax.ShapeDtypeStruct((2 * R_BIG * _TO[2], H), jnp.float32),
    ]
    if layer0:
        out_type.append(jax.ShapeDtypeStruct((2 * R_TAG, H), jnp.float32))
        out_type += [
            jax.ShapeDtypeStruct((2 * R_BIG * _RATES[2],), jnp.float32),
            jax.ShapeDtypeStruct((2 * R_BIG * _REV[2],), jnp.float32),
            jax.ShapeDtypeStruct((2 * R_BIG * _TO[2],), jnp.float32),
            jax.ShapeDtypeStruct((2 * R_TAG,), jnp.float32),
        ]

    scratch = [
        pltpu.VMEM_SHARED((ACC_ROWS, H), jnp.float32),
        pltpu.VMEM_SHARED((ACC_ROWS,), jnp.float32),
        pltpu.VMEM((CH,), jnp.int32),
        pltpu.VMEM((CH,), jnp.int32),
        pltpu.VMEM((512,), jnp.int32),
        pltpu.VMEM((512,), jnp.int32),
        pltpu.VMEM((GB,), jnp.int32),
        pltpu.VMEM((GB,), jnp.int32),
        pltpu.VMEM((GB,), jnp.int32),
        pltpu.VMEM((GB,), jnp.int32),
        pltpu.VMEM((GB, H), jnp.float32),
        pltpu.VMEM((GB, H), jnp.float32),
        pltpu.VMEM((GB,), jnp.float32),
        pltpu.VMEM((GB,), jnp.float32),
        pltpu.VMEM((GB,), jnp.float32),
        pltpu.SemaphoreType.DMA,
        pltpu.SemaphoreType.DMA,
    ]

    def body(h_user, h_movie, h_tag,
             r_src, r_dst, v_src, v_dst, o_src, o_dst, ht_src, ht_dst,
             ones1h, zer, zer1, *rest):
        if layer0:
            (s_rates, s_rev, s_to, s_ht, d_rates, d_rev, d_to, d_ht,
             acc, dacc, src_ch, dst_ch, cbs, cbd, fsrcA, fdstA, fsrcB,
             fdstB, rowsA, rowsB, cpb1, z1, ones_v, semA, semB) = rest
        else:
            (s_rates, s_rev, s_to,
             acc, dacc, src_ch, dst_ch, cbs, cbd, fsrcA, fdstA, fsrcB,
             fdstB, rowsA, rowsB, cpb1, z1, ones_v, semA, semB) = rest
            d_rates = d_rev = d_to = d_ht = s_ht = None
        c = lax.axis_index("c")
        s = lax.axis_index("s")
        pltpu.sync_copy(ones1h, ones_v)
        pltpu.sync_copy(zer1, z1)
        common = dict(zer=zer, acc=acc, dacc=dacc, src_ch=src_ch,
                      dst_ch=dst_ch, cbs=cbs, cbd=cbd, fsrcA=fsrcA,
                      fdstA=fdstA, fsrcB=fsrcB, fdstB=fdstB, rowsA=rowsA,
                      rowsB=rowsB, cpb1=cpb1, z1=z1, ones_v=ones_v,
                      semA=semA, semB=semB)
        _run_agg(c, s, h_user, r_src, r_dst, EPR, _RATES[1], _RATES[2],
                 s_rates, d_rates, **common)
        _run_agg(c, s, h_movie, v_src, v_dst, EPR, _REV[1], _REV[2],
                 s_rev, d_rev, **common)
        _run_agg(c, s, h_tag, o_src, o_dst, EPT, _TO[1], _TO[2],
                 s_to, d_to, **common)
        if layer0:
            _run_agg(c, s, h_movie, ht_src, ht_dst, EPT, _HT[1], _HT[2],
                     s_ht, d_ht, **common)

    return pl.kernel(body, out_type=tuple(out_type), mesh=mesh,
                     scratch_types=scratch,
                     compiler_params=pltpu.CompilerParams(
                         needs_layout_passes=False))


_agg_cache = {}


def _agg(layer0, *args):
    if layer0 not in _agg_cache:
        _agg_cache[layer0] = _make_agg_kernel(layer0)
    return _agg_cache[layer0](*args)


_agg_l0 = functools.partial(_agg, True)
_agg_l1 = functools.partial(_agg, False)


def _proj_movie(movie_genre_p, W_proj, b_proj):
    def body(g_ref, w_ref, b_ref, o_ref):
        o_ref[...] = jnp.dot(g_ref[...], w_ref[...],
                             preferred_element_type=jnp.float32) + b_ref[...]

    return pl.pallas_call(
        body,
        grid=(NMP // BK,),
        in_specs=[pl.BlockSpec((BK, D_FEAT), lambda i: (i, 0)),
                  pl.BlockSpec((D_FEAT, H), lambda i: (0, 0)),
                  pl.BlockSpec((1, H), lambda i: (0, 0))],
        out_specs=pl.BlockSpec((BK, H), lambda i: (i, 0)),
        out_shape=jax.ShapeDtypeStruct((NMP, H), jnp.float32),
    )(movie_genre_p, W_proj, b_proj.reshape(1, H))


def _eye128():
    ri = lax.broadcasted_iota(jnp.int32, (H, H), 0)
    ci = lax.broadcasted_iota(jnp.int32, (H, H), 1)
    return (ri == ci).astype(jnp.float32)


def _dcol(eye, d2, b):
    drow = d2[b:b + 1, :]
    return lax.dot_general(eye, drow, (((1,), (1,)), ((), ())),
                           preferred_element_type=jnp.float32)


def _dense1(h, s_n, deg2, Ws, Wn, b):
    n = h.shape[0]

    def body(h_ref, s_ref, d_ref, ws_ref, wn_ref, b_ref, o_ref):
        eye = _eye128()
        x = jnp.dot(h_ref[...], ws_ref[...],
                    preferred_element_type=jnp.float32) + b_ref[...]
        for bb in range(BK // H):
            dcol = _dcol(eye, d_ref[...], bb)
            mb = s_ref[pl.ds(bb * H, H), :] * (1.0 / jnp.maximum(dcol, 1.0))
            zb = x[bb * H:(bb + 1) * H, :] + jnp.dot(
                mb, wn_ref[...], preferred_element_type=jnp.float32)
            o_ref[pl.ds(bb * H, H), :] = jnp.where(zb >= 0, zb, 0.1 * zb)

    row = lambda i: (i, 0)
    full = lambda i: (0, 0)
    return pl.pallas_call(
        body,
        grid=(n // BK,),
        in_specs=[pl.BlockSpec((BK, H), row),
                  pl.BlockSpec((BK, H), row),
                  pl.BlockSpec((BK // H, H), row),
                  pl.BlockSpec((H, H), full),
                  pl.BlockSpec((H, H), full),
                  pl.BlockSpec((1, H), full)],
        out_specs=pl.BlockSpec((BK, H), row),
        out_shape=jax.ShapeDtypeStruct((n, H), jnp.float32),
    )(h, s_n, deg2, Ws, Wn, b.reshape(1, H))


def _dense2(h, s1, d1, s2, d2, Wsa, Wsb, Wn1, Wn2, b):
    n = h.shape[0]

    def body(h_ref, s1_ref, d1_ref, s2_ref, d2_ref,
             wsa_ref, wsb_ref, wn1_ref, wn2_ref, b_ref, o_ref):
        eye = _eye128()
        ws = wsa_ref[...] + wsb_ref[...]
        x = jnp.dot(h_ref[...], ws,
                    preferred_element_type=jnp.float32) + b_ref[...]
        for bb in range(BK // H):
            c1 = _dcol(eye, d1_ref[...], bb)
            c2 = _dcol(eye, d2_ref[...], bb)
            m1 = s1_ref[pl.ds(bb * H, H), :] * (1.0 / jnp.maximum(c1, 1.0))
            m2 = s2_ref[pl.ds(bb * H, H), :] * (1.0 / jnp.maximum(c2, 1.0))
            zb = x[bb * H:(bb + 1) * H, :]
            zb = zb + jnp.dot(m1, wn1_ref[...], preferred_element_type=jnp.float32)
            zb = zb + jnp.dot(m2, wn2_ref[...], preferred_element_type=jnp.float32)
            o_ref[pl.ds(bb * H, H), :] = jnp.where(zb >= 0, zb, 0.1 * zb)

    row = lambda i: (i, 0)
    full = lambda i: (0, 0)
    return pl.pallas_call(
        body,
        grid=(n // BK,),
        in_specs=[pl.BlockSpec((BK, H), row),
                  pl.BlockSpec((BK, H), row),
                  pl.BlockSpec((BK // H, H), row),
                  pl.BlockSpec((BK, H), row),
                  pl.BlockSpec((BK // H, H), row),
                  pl.BlockSpec((H, H), full),
                  pl.BlockSpec((H, H), full),
                  pl.BlockSpec((H, H), full),
                  pl.BlockSpec((H, H), full),
                  pl.BlockSpec((1, H), full)],
        out_specs=pl.BlockSpec((BK, H), row),
        out_shape=jax.ShapeDtypeStruct((n, H), jnp.float32),
    )(h, s1, d1, s2, d2, Wsa, Wsb, Wn1, Wn2, b.reshape(1, H))


def _pad_edges(idx, epad, fill):
    pad = jnp.full((epad - idx.shape[0],), fill, dtype=jnp.int32)
    return jnp.concatenate([idx.astype(jnp.int32), pad])


def _pad_rows(x, n):
    return jnp.pad(x, ((0, n - x.shape[0]), (0, 0)))


def _deg2(d_flat, npadded):
    return d_flat.reshape(-1, H)[: npadded // H]


@jax.jit
def kernel(movie_genre, rates_src, rates_dst, has_tag_src, has_tag_dst,
           user_emb, tag_emb, W_proj, b_proj, W_self, W_neigh, b_conv):
    r_src = _pad_edges(rates_src, EPR, 0)
    r_dst = _pad_edges(rates_dst, EPR, NM)
    v_src = _pad_edges(rates_dst, EPR, 0)
    v_dst = _pad_edges(rates_src, EPR, NU)
    ht_src = _pad_edges(has_tag_src, EPT, 0)
    ht_dst = _pad_edges(has_tag_dst, EPT, NT)
    o_src = _pad_edges(has_tag_dst, EPT, 0)
    o_dst = _pad_edges(has_tag_src, EPT, NM)

    ones1 = jnp.ones((GB,), jnp.float32)
    zer = jnp.zeros((GB, H), jnp.float32)
    zer1 = jnp.zeros((GB,), jnp.float32)

    h_u0 = _pad_rows(user_emb, NUP)
    h_t0 = _pad_rows(tag_emb, NTP)
    h_m0 = _proj_movie(_pad_rows(movie_genre, NMP), W_proj, b_proj)

    (s_rates, s_rev, s_to, s_ht, d_rates, d_rev, d_to, d_ht) = _agg_l0(
        h_u0, h_m0, h_t0,
        r_src, r_dst, v_src, v_dst, o_src, o_dst, ht_src, ht_dst,
        ones1, zer, zer1)

    d2_rates = _deg2(d_rates, NMP)
    d2_rev = _deg2(d_rev, NUP)
    d2_to = _deg2(d_to, NMP)
    d2_ht = _deg2(d_ht, NTP)
    h_u1 = _dense1(h_u0, s_rev[:NUP], d2_rev,
                   W_self[0, 1], W_neigh[0, 1], b_conv[0, 1])
    h_t1 = _dense1(h_t0, s_ht[:NTP], d2_ht,
                   W_self[0, 2], W_neigh[0, 2], b_conv[0, 2])
    h_m1 = _dense2(h_m0, s_rates[:NMP], d2_rates, s_to[:NMP], d2_to,
                   W_self[0, 0], W_self[0, 3], W_neigh[0, 0], W_neigh[0, 3],
                   b_conv[0, 0] + b_conv[0, 3])

    (s_rates1, s_rev1, s_to1) = _agg_l1(
        h_u1, h_m1, h_t1,
        r_src, r_dst, v_src, v_dst, o_src, o_dst, ht_src, ht_dst,
        ones1, zer, zer1)

    h_u2 = _dense1(h_u1, s_rev1[:NUP], d2_rev,
                   W_self[1, 1], W_neigh[1, 1], b_conv[1, 1])
    h_m2 = _dense2(h_m1, s_rates1[:NMP], d2_rates, s_to1[:NMP], d2_to,
                   W_self[1, 0], W_self[1, 3], W_neigh[1, 0], W_neigh[1, 3],
                   b_conv[1, 0] + b_conv[1, 3])
    return (h_u2[:NU], h_m2[:NM])


# speedup vs baseline: 1.0858x; 1.0522x over previous
import functools

import jax
import jax.numpy as jnp
from jax import lax
from jax.experimental import pallas as pl
from jax.experimental.pallas import tpu as pltpu
from jax.experimental.pallas import tpu_sc as plsc

NU, NM, NT = 100000, 50000, 5000
D_FEAT, H = 20, 128
BK = 1024
NUP, NMP, NTP = 100352, 50176, 5120

CH = 2048
GB = 128
R_BIG = 8192
R_TAG = 4096
ACC_ROWS = R_BIG + 128
EPR = 524288
EPT = 131072

_RATES = (NM, R_BIG, 4)
_REV = (NU, R_BIG, 7)
_HT = (NT, R_TAG, 1)
_TO = (NM, R_BIG, 4)


def _run_agg(c, s, h_tbl, src_e, dst_e, epad, R, npass, s_out, d_out, zer,
             acc, dacc, src_ch, dst_ch, cbs, cbd, fsrcA, fdstA, fsrcB,
             fdstB, rowsA, rowsB, cpb1, z1, ones_v, semA, semB,
             semS, semD):
    nchunks = epad // 16 // CH
    tile_base = s * (epad // 16)
    nzb = (R + 128) // 128
    ncp = R // 128
    do_deg = d_out is not None
    rows = rowsA

    def stage(base, fsrc, fdst):
        for j in range(8):
            fsrc[pl.ds(j * 16, 16)] = cbs[pl.ds(base + j * 16, 16)]
            fdst[pl.ds(j * 16, 16)] = cbd[pl.ds(base + j * 16, 16)]

    def wait_scatters():
        pltpu.make_async_copy(rowsA, acc.at[fdstA], semS).wait()
        pltpu.make_async_copy(rowsB, acc.at[fdstB], semS).wait()
        if do_deg:
            pltpu.make_async_copy(ones_v, dacc.at[fdstA], semD).wait()
            pltpu.make_async_copy(ones_v, dacc.at[fdstB], semD).wait()

    def issue_scatter(fdst, rows_s):
        pltpu.async_copy(rows_s, acc.at[fdst], semS, add=True)
        if do_deg:
            pltpu.async_copy(ones_v, dacc.at[fdst], semD, add=True)

    def wait_tail(fdst):
        pltpu.make_async_copy(rowsA, acc.at[fdst], semS).wait()
        if do_deg:
            pltpu.make_async_copy(ones_v, dacc.at[fdst], semD).wait()

    def flush2(pend):
        @pl.when(pend == 1)
        def _():
            wait_scatters()
        stage(0, fsrcA, fdstA)
        stage(GB, fsrcB, fdstB)
        ga = pltpu.async_copy(h_tbl.at[fsrcA], rowsA, semA)
        gb = pltpu.async_copy(h_tbl.at[fsrcB], rowsB, semB)
        ga.wait()
        issue_scatter(fdstA, rowsA)
        gb.wait()
        issue_scatter(fdstB, rowsB)

    def pass_body(p, _):
        lo = p * (2 * R) + c * R
        pltpu.sync_copy(zer, rows)

        def zero_blk(j, _):
            blk = s + j * 16

            @pl.when(blk < nzb)
            def _():
                pltpu.sync_copy(rows, acc.at[pl.ds(blk * 128, 128)])
                if do_deg:
                    pltpu.sync_copy(z1, dacc.at[pl.ds(blk * 128, 128)])
            return 0

        lax.fori_loop(0, -(-nzb // 16), zero_blk, 0)
        plsc.subcore_barrier()

        def chunk_body(ci, carry):
            eb = tile_base + ci * CH
            pltpu.sync_copy(src_e.at[pl.ds(eb, CH)], src_ch)
            pltpu.sync_copy(dst_e.at[pl.ds(eb, CH)], dst_ch)

            def vec_body(v, carry):
                cnt, pend = carry
                sv = src_ch[pl.ds(v * 16, 16)]
                dv = dst_ch[pl.ds(v * 16, 16)]
                m = (dv >= lo) & (dv < lo + R)
                mi = m.astype(jnp.int32)
                pos = jnp.maximum(cnt + plsc.cumsum(mi) - 1, 0)
                plsc.store_scatter(cbs, [pos], sv, mask=m)
                plsc.store_scatter(cbd, [pos], dv - lo, mask=m)
                cnt = cnt + jnp.sum(mi)

                @pl.when(cnt >= 2 * GB)
                def _():
                    flush2(pend)
                    for j in range(16):
                        cbs[pl.ds(j * 16, 16)] = cbs[pl.ds(2 * GB + j * 16, 16)]
                        cbd[pl.ds(j * 16, 16)] = cbd[pl.ds(2 * GB + j * 16, 16)]

                fired = (cnt >= 2 * GB).astype(jnp.int32)
                pend = jnp.maximum(pend, fired)
                return (jnp.where(cnt >= 2 * GB, cnt - 2 * GB, cnt), pend)

            return lax.fori_loop(0, CH // 16, vec_body, carry)

        cnt, pend = lax.fori_loop(0, nchunks, chunk_body,
                                  (jnp.int32(0), jnp.int32(0)))

        @pl.when(pend == 1)
        def _():
            wait_scatters()

        iota = lax.iota(jnp.int32, 16)
        for j in range(16):
            lane = iota + j * 16
            sv = cbs[pl.ds(j * 16, 16)]
            dv = cbd[pl.ds(j * 16, 16)]
            keep = lane < cnt
            cbs[pl.ds(j * 16, 16)] = jnp.where(keep, sv, 0)
            cbd[pl.ds(j * 16, 16)] = jnp.where(keep, dv, R)

        @pl.when(cnt > 0)
        def _():
            stage(0, fsrcA, fdstA)
            pltpu.async_copy(h_tbl.at[fsrcA], rowsA, semA).wait()
            issue_scatter(fdstA, rowsA)
            wait_tail(fdstA)

        @pl.when(cnt > GB)
        def _():
            stage(GB, fsrcB, fdstB)
            pltpu.async_copy(h_tbl.at[fsrcB], rowsB, semB).wait()
            issue_scatter(fdstB, rowsB)
            wait_tail(fdstB)

        plsc.subcore_barrier()
        def cp_blk(j, _):
            blk = s + j * 16
            pltpu.sync_copy(acc.at[pl.ds(blk * 128, 128)], rows)
            pltpu.sync_copy(rows, s_out.at[pl.ds(lo + blk * 128, 128)])
            if do_deg:
                pltpu.sync_copy(dacc.at[pl.ds(blk * 128, 128)], cpb1)
                pltpu.sync_copy(cpb1, d_out.at[pl.ds(lo + blk * 128, 128)])
            return 0

        lax.fori_loop(0, ncp // 16, cp_blk, 0)
        plsc.subcore_barrier()
        return 0

    lax.fori_loop(0, npass, pass_body, 0)


def _make_agg_kernel(layer0: bool):
    mesh = plsc.VectorSubcoreMesh(core_axis_name="c", subcore_axis_name="s",
                                  num_cores=2, num_subcores=16)

    out_type = [
        jax.ShapeDtypeStruct((2 * R_BIG * _RATES[2], H), jnp.float32),
        jax.ShapeDtypeStruct((2 * R_BIG * _REV[2], H), jnp.float32),
        jax.ShapeDtypeStruct((2 * R_BIG * _TO[2], H), jnp.float32),
    ]
    if layer0:
        out_type.append(jax.ShapeDtypeStruct((2 * R_TAG, H), jnp.float32))
        out_type += [
            jax.ShapeDtypeStruct((2 * R_BIG * _RATES[2],), jnp.float32),
            jax.ShapeDtypeStruct((2 * R_BIG * _REV[2],), jnp.float32),
            jax.ShapeDtypeStruct((2 * R_BIG * _TO[2],), jnp.float32),
            jax.ShapeDtypeStruct((2 * R_TAG,), jnp.float32),
        ]

    scratch = [
        pltpu.VMEM_SHARED((ACC_ROWS, H), jnp.float32),
        pltpu.VMEM_SHARED((ACC_ROWS,), jnp.float32),
        pltpu.VMEM((CH,), jnp.int32),
        pltpu.VMEM((CH,), jnp.int32),
        pltpu.VMEM((512,), jnp.int32),
        pltpu.VMEM((512,), jnp.int32),
        pltpu.VMEM((GB,), jnp.int32),
        pltpu.VMEM((GB,), jnp.int32),
        pltpu.VMEM((GB,), jnp.int32),
        pltpu.VMEM((GB,), jnp.int32),
        pltpu.VMEM((GB, H), jnp.float32),
        pltpu.VMEM((GB, H), jnp.float32),
        pltpu.VMEM((GB,), jnp.float32),
        pltpu.VMEM((GB,), jnp.float32),
        pltpu.VMEM((GB,), jnp.float32),
        pltpu.SemaphoreType.DMA,
        pltpu.SemaphoreType.DMA,
        pltpu.SemaphoreType.DMA,
        pltpu.SemaphoreType.DMA,
    ]

    def body(h_user, h_movie, h_tag,
             r_src, r_dst, v_src, v_dst, o_src, o_dst, ht_src, ht_dst,
             ones1h, zer, zer1, *rest):
        if layer0:
            (s_rates, s_rev, s_to, s_ht, d_rates, d_rev, d_to, d_ht,
             acc, dacc, src_ch, dst_ch, cbs, cbd, fsrcA, fdstA, fsrcB,
             fdstB, rowsA, rowsB, cpb1, z1, ones_v, semA, semB,
             semS, semD) = rest
        else:
            (s_rates, s_rev, s_to,
             acc, dacc, src_ch, dst_ch, cbs, cbd, fsrcA, fdstA, fsrcB,
             fdstB, rowsA, rowsB, cpb1, z1, ones_v, semA, semB,
             semS, semD) = rest
            d_rates = d_rev = d_to = d_ht = s_ht = None
        c = lax.axis_index("c")
        s = lax.axis_index("s")
        pltpu.sync_copy(ones1h, ones_v)
        pltpu.sync_copy(zer1, z1)
        common = dict(zer=zer, acc=acc, dacc=dacc, src_ch=src_ch,
                      dst_ch=dst_ch, cbs=cbs, cbd=cbd, fsrcA=fsrcA,
                      fdstA=fdstA, fsrcB=fsrcB, fdstB=fdstB, rowsA=rowsA,
                      rowsB=rowsB, cpb1=cpb1, z1=z1, ones_v=ones_v,
                      semA=semA, semB=semB, semS=semS, semD=semD)
        _run_agg(c, s, h_user, r_src, r_dst, EPR, _RATES[1], _RATES[2],
                 s_rates, d_rates, **common)
        _run_agg(c, s, h_movie, v_src, v_dst, EPR, _REV[1], _REV[2],
                 s_rev, d_rev, **common)
        _run_agg(c, s, h_tag, o_src, o_dst, EPT, _TO[1], _TO[2],
                 s_to, d_to, **common)
        if layer0:
            _run_agg(c, s, h_movie, ht_src, ht_dst, EPT, _HT[1], _HT[2],
                     s_ht, d_ht, **common)

    return pl.kernel(body, out_type=tuple(out_type), mesh=mesh,
                     scratch_types=scratch,
                     compiler_params=pltpu.CompilerParams(
                         needs_layout_passes=False))


_agg_cache = {}


def _agg(layer0, *args):
    if layer0 not in _agg_cache:
        _agg_cache[layer0] = _make_agg_kernel(layer0)
    return _agg_cache[layer0](*args)


_agg_l0 = functools.partial(_agg, True)
_agg_l1 = functools.partial(_agg, False)


def _proj_movie(movie_genre_p, W_proj, b_proj):
    def body(g_ref, w_ref, b_ref, o_ref):
        o_ref[...] = jnp.dot(g_ref[...], w_ref[...],
                             preferred_element_type=jnp.float32) + b_ref[...]

    return pl.pallas_call(
        body,
        grid=(NMP // BK,),
        in_specs=[pl.BlockSpec((BK, D_FEAT), lambda i: (i, 0)),
                  pl.BlockSpec((D_FEAT, H), lambda i: (0, 0)),
                  pl.BlockSpec((1, H), lambda i: (0, 0))],
        out_specs=pl.BlockSpec((BK, H), lambda i: (i, 0)),
        out_shape=jax.ShapeDtypeStruct((NMP, H), jnp.float32),
    )(movie_genre_p, W_proj, b_proj.reshape(1, H))


def _eye128():
    ri = lax.broadcasted_iota(jnp.int32, (H, H), 0)
    ci = lax.broadcasted_iota(jnp.int32, (H, H), 1)
    return (ri == ci).astype(jnp.float32)


def _dcol(eye, d2, b):
    drow = d2[b:b + 1, :]
    return lax.dot_general(eye, drow, (((1,), (1,)), ((), ())),
                           preferred_element_type=jnp.float32)


def _dense1(h, s_n, deg2, Ws, Wn, b):
    n = h.shape[0]

    def body(h_ref, s_ref, d_ref, ws_ref, wn_ref, b_ref, o_ref):
        eye = _eye128()
        x = jnp.dot(h_ref[...], ws_ref[...],
                    preferred_element_type=jnp.float32) + b_ref[...]
        for bb in range(BK // H):
            dcol = _dcol(eye, d_ref[...], bb)
            mb = s_ref[pl.ds(bb * H, H), :] * (1.0 / jnp.maximum(dcol, 1.0))
            zb = x[bb * H:(bb + 1) * H, :] + jnp.dot(
                mb, wn_ref[...], preferred_element_type=jnp.float32)
            o_ref[pl.ds(bb * H, H), :] = jnp.where(zb >= 0, zb, 0.1 * zb)

    row = lambda i: (i, 0)
    full = lambda i: (0, 0)
    return pl.pallas_call(
        body,
        grid=(n // BK,),
        in_specs=[pl.BlockSpec((BK, H), row),
                  pl.BlockSpec((BK, H), row),
                  pl.BlockSpec((BK // H, H), row),
                  pl.BlockSpec((H, H), full),
                  pl.BlockSpec((H, H), full),
                  pl.BlockSpec((1, H), full)],
        out_specs=pl.BlockSpec((BK, H), row),
        out_shape=jax.ShapeDtypeStruct((n, H), jnp.float32),
    )(h, s_n, deg2, Ws, Wn, b.reshape(1, H))


def _dense2(h, s1, d1, s2, d2, Wsa, Wsb, Wn1, Wn2, b):
    n = h.shape[0]

    def body(h_ref, s1_ref, d1_ref, s2_ref, d2_ref,
             wsa_ref, wsb_ref, wn1_ref, wn2_ref, b_ref, o_ref):
        eye = _eye128()
        ws = wsa_ref[...] + wsb_ref[...]
        x = jnp.dot(h_ref[...], ws,
                    preferred_element_type=jnp.float32) + b_ref[...]
        for bb in range(BK // H):
            c1 = _dcol(eye, d1_ref[...], bb)
            c2 = _dcol(eye, d2_ref[...], bb)
            m1 = s1_ref[pl.ds(bb * H, H), :] * (1.0 / jnp.maximum(c1, 1.0))
            m2 = s2_ref[pl.ds(bb * H, H), :] * (1.0 / jnp.maximum(c2, 1.0))
            zb = x[bb * H:(bb + 1) * H, :]
            zb = zb + jnp.dot(m1, wn1_ref[...], preferred_element_type=jnp.float32)
            zb = zb + jnp.dot(m2, wn2_ref[...], preferred_element_type=jnp.float32)
            o_ref[pl.ds(bb * H, H), :] = jnp.where(zb >= 0, zb, 0.1 * zb)

    row = lambda i: (i, 0)
    full = lambda i: (0, 0)
    return pl.pallas_call(
        body,
        grid=(n // BK,),
        in_specs=[pl.BlockSpec((BK, H), row),
                  pl.BlockSpec((BK, H), row),
                  pl.BlockSpec((BK // H, H), row),
                  pl.BlockSpec((BK, H), row),
                  pl.BlockSpec((BK // H, H), row),
                  pl.BlockSpec((H, H), full),
                  pl.BlockSpec((H, H), full),
                  pl.BlockSpec((H, H), full),
                  pl.BlockSpec((H, H), full),
                  pl.BlockSpec((1, H), full)],
        out_specs=pl.BlockSpec((BK, H), row),
        out_shape=jax.ShapeDtypeStruct((n, H), jnp.float32),
    )(h, s1, d1, s2, d2, Wsa, Wsb, Wn1, Wn2, b.reshape(1, H))


def _pad_edges(idx, epad, fill):
    pad = jnp.full((epad - idx.shape[0],), fill, dtype=jnp.int32)
    return jnp.concatenate([idx.astype(jnp.int32), pad])


def _pad_rows(x, n):
    return jnp.pad(x, ((0, n - x.shape[0]), (0, 0)))


def _deg2(d_flat, npadded):
    return d_flat.reshape(-1, H)[: npadded // H]


@jax.jit
def kernel(movie_genre, rates_src, rates_dst, has_tag_src, has_tag_dst,
           user_emb, tag_emb, W_proj, b_proj, W_self, W_neigh, b_conv):
    r_src = _pad_edges(rates_src, EPR, 0)
    r_dst = _pad_edges(rates_dst, EPR, NM)
    v_src = _pad_edges(rates_dst, EPR, 0)
    v_dst = _pad_edges(rates_src, EPR, NU)
    ht_src = _pad_edges(has_tag_src, EPT, 0)
    ht_dst = _pad_edges(has_tag_dst, EPT, NT)
    o_src = _pad_edges(has_tag_dst, EPT, 0)
    o_dst = _pad_edges(has_tag_src, EPT, NM)

    ones1 = jnp.ones((GB,), jnp.float32)
    zer = jnp.zeros((GB, H), jnp.float32)
    zer1 = jnp.zeros((GB,), jnp.float32)

    h_u0 = _pad_rows(user_emb, NUP)
    h_t0 = _pad_rows(tag_emb, NTP)
    h_m0 = _proj_movie(_pad_rows(movie_genre, NMP), W_proj, b_proj)

    (s_rates, s_rev, s_to, s_ht, d_rates, d_rev, d_to, d_ht) = _agg_l0(
        h_u0, h_m0, h_t0,
        r_src, r_dst, v_src, v_dst, o_src, o_dst, ht_src, ht_dst,
        ones1, zer, zer1)

    d2_rates = _deg2(d_rates, NMP)
    d2_rev = _deg2(d_rev, NUP)
    d2_to = _deg2(d_to, NMP)
    d2_ht = _deg2(d_ht, NTP)
    h_u1 = _dense1(h_u0, s_rev[:NUP], d2_rev,
                   W_self[0, 1], W_neigh[0, 1], b_conv[0, 1])
    h_t1 = _dense1(h_t0, s_ht[:NTP], d2_ht,
                   W_self[0, 2], W_neigh[0, 2], b_conv[0, 2])
    h_m1 = _dense2(h_m0, s_rates[:NMP], d2_rates, s_to[:NMP], d2_to,
                   W_self[0, 0], W_self[0, 3], W_neigh[0, 0], W_neigh[0, 3],
                   b_conv[0, 0] + b_conv[0, 3])

    (s_rates1, s_rev1, s_to1) = _agg_l1(
        h_u1, h_m1, h_t1,
        r_src, r_dst, v_src, v_dst, o_src, o_dst, ht_src, ht_dst,
        ones1, zer, zer1)

    h_u2 = _dense1(h_u1, s_rev1[:NUP], d2_rev,
                   W_self[1, 1], W_neigh[1, 1], b_conv[1, 1])
    h_m2 = _dense2(h_m1, s_rates1[:NMP], d2_rates, s_to1[:NMP], d2_to,
                   W_self[1, 0], W_self[1, 3], W_neigh[1, 0], W_neigh[1, 3],
                   b_conv[1, 0] + b_conv[1, 3])
    return (h_u2[:NU], h_m2[:NM])


# speedup vs baseline: 1.1171x; 1.0289x over previous
import functools

import jax
import jax.numpy as jnp
from jax import lax
from jax.experimental import pallas as pl
from jax.experimental.pallas import tpu as pltpu
from jax.experimental.pallas import tpu_sc as plsc

NU, NM, NT = 100000, 50000, 5000
D_FEAT, H = 20, 128
BK = 1024
NUP, NMP, NTP = 100352, 50176, 5120

CH = 2048
GB = 128
R_BIG = 8192
R_TAG = 4096
ACC_ROWS = R_BIG + 128
EPR = 524288
EPT = 131072

_RATES = (NM, R_BIG, 4)
_REV = (NU, R_BIG, 7)
_HT = (NT, R_TAG, 1)
_TO = (NM, R_BIG, 4)


def _run_agg(c, s, h_tbl, src_e, dst_e, epad, R, npass, s_out, d_out, zer,
             acc, dacc, src_ch, dst_ch, src_ch2, dst_ch2, semI, semI2,
             cbs, cbd, fsrcA, fdstA, fsrcB,
             fdstB, rowsA, rowsB, cpb1, z1, ones_v, semA, semB,
             semS, semD):
    nchunks = epad // 16 // CH
    tile_base = s * (epad // 16)
    nzb = (R + 128) // 128
    ncp = R // 128
    do_deg = d_out is not None
    rows = rowsA

    def stage(base, fsrc, fdst):
        for j in range(8):
            fsrc[pl.ds(j * 16, 16)] = cbs[pl.ds(base + j * 16, 16)]
            fdst[pl.ds(j * 16, 16)] = cbd[pl.ds(base + j * 16, 16)]

    def wait_scatters():
        pltpu.make_async_copy(rowsA, acc.at[fdstA], semS).wait()
        pltpu.make_async_copy(rowsB, acc.at[fdstB], semS).wait()
        if do_deg:
            pltpu.make_async_copy(ones_v, dacc.at[fdstA], semD).wait()
            pltpu.make_async_copy(ones_v, dacc.at[fdstB], semD).wait()

    def issue_scatter(fdst, rows_s):
        pltpu.async_copy(rows_s, acc.at[fdst], semS, add=True)
        if do_deg:
            pltpu.async_copy(ones_v, dacc.at[fdst], semD, add=True)

    def wait_tail(fdst):
        pltpu.make_async_copy(rowsA, acc.at[fdst], semS).wait()
        if do_deg:
            pltpu.make_async_copy(ones_v, dacc.at[fdst], semD).wait()

    def flush2(pend):
        @pl.when(pend == 1)
        def _():
            wait_scatters()
        stage(0, fsrcA, fdstA)
        stage(GB, fsrcB, fdstB)
        ga = pltpu.async_copy(h_tbl.at[fsrcA], rowsA, semA)
        gb = pltpu.async_copy(h_tbl.at[fsrcB], rowsB, semB)
        ga.wait()
        issue_scatter(fdstA, rowsA)
        gb.wait()
        issue_scatter(fdstB, rowsB)

    def pass_body(p, _):
        lo = p * (2 * R) + c * R
        pltpu.sync_copy(zer, rows)

        def zero_blk(j, _):
            blk = s + j * 16

            @pl.when(blk < nzb)
            def _():
                pltpu.sync_copy(rows, acc.at[pl.ds(blk * 128, 128)])
                if do_deg:
                    pltpu.sync_copy(z1, dacc.at[pl.ds(blk * 128, 128)])
            return 0

        lax.fori_loop(0, -(-nzb // 16), zero_blk, 0)
        plsc.subcore_barrier()

        def chunk_body(ci, carry):
            eb = tile_base + (2 * ci) * CH
            eb2 = eb + CH
            g1 = pltpu.async_copy(src_e.at[pl.ds(eb, CH)], src_ch, semI)
            g2 = pltpu.async_copy(dst_e.at[pl.ds(eb, CH)], dst_ch, semI)
            g3 = pltpu.async_copy(src_e.at[pl.ds(eb2, CH)], src_ch2, semI2)
            g4 = pltpu.async_copy(dst_e.at[pl.ds(eb2, CH)], dst_ch2, semI2)
            g1.wait()
            g2.wait()

            def vec_body2(v, carry):
                cnt, pend = carry
                sv = src_ch2[pl.ds(v * 16, 16)]
                dv = dst_ch2[pl.ds(v * 16, 16)]
                m = (dv >= lo) & (dv < lo + R)
                mi = m.astype(jnp.int32)
                pos = jnp.maximum(cnt + plsc.cumsum(mi) - 1, 0)
                plsc.store_scatter(cbs, [pos], sv, mask=m)
                plsc.store_scatter(cbd, [pos], dv - lo, mask=m)
                cnt = cnt + jnp.sum(mi)

                @pl.when(cnt >= 2 * GB)
                def _():
                    flush2(pend)
                    for j in range(16):
                        cbs[pl.ds(j * 16, 16)] = cbs[pl.ds(2 * GB + j * 16, 16)]
                        cbd[pl.ds(j * 16, 16)] = cbd[pl.ds(2 * GB + j * 16, 16)]

                fired = (cnt >= 2 * GB).astype(jnp.int32)
                pend = jnp.maximum(pend, fired)
                return (jnp.where(cnt >= 2 * GB, cnt - 2 * GB, cnt), pend)

            def vec_body(v, carry):
                cnt, pend = carry
                sv = src_ch[pl.ds(v * 16, 16)]
                dv = dst_ch[pl.ds(v * 16, 16)]
                m = (dv >= lo) & (dv < lo + R)
                mi = m.astype(jnp.int32)
                pos = jnp.maximum(cnt + plsc.cumsum(mi) - 1, 0)
                plsc.store_scatter(cbs, [pos], sv, mask=m)
                plsc.store_scatter(cbd, [pos], dv - lo, mask=m)
                cnt = cnt + jnp.sum(mi)

                @pl.when(cnt >= 2 * GB)
                def _():
                    flush2(pend)
                    for j in range(16):
                        cbs[pl.ds(j * 16, 16)] = cbs[pl.ds(2 * GB + j * 16, 16)]
                        cbd[pl.ds(j * 16, 16)] = cbd[pl.ds(2 * GB + j * 16, 16)]

                fired = (cnt >= 2 * GB).astype(jnp.int32)
                pend = jnp.maximum(pend, fired)
                return (jnp.where(cnt >= 2 * GB, cnt - 2 * GB, cnt), pend)

            carry = lax.fori_loop(0, CH // 16, vec_body, carry)
            g3.wait()
            g4.wait()
            return lax.fori_loop(0, CH // 16, vec_body2, carry)

        cnt, pend = lax.fori_loop(0, nchunks // 2, chunk_body,
                                  (jnp.int32(0), jnp.int32(0)))

        @pl.when(pend == 1)
        def _():
            wait_scatters()

        iota = lax.iota(jnp.int32, 16)
        for j in range(16):
            lane = iota + j * 16
            sv = cbs[pl.ds(j * 16, 16)]
            dv = cbd[pl.ds(j * 16, 16)]
            keep = lane < cnt
            cbs[pl.ds(j * 16, 16)] = jnp.where(keep, sv, 0)
            cbd[pl.ds(j * 16, 16)] = jnp.where(keep, dv, R)

        @pl.when(cnt > 0)
        def _():
            stage(0, fsrcA, fdstA)
            pltpu.async_copy(h_tbl.at[fsrcA], rowsA, semA).wait()
            issue_scatter(fdstA, rowsA)
            wait_tail(fdstA)

        @pl.when(cnt > GB)
        def _():
            stage(GB, fsrcB, fdstB)
            pltpu.async_copy(h_tbl.at[fsrcB], rowsB, semB).wait()
            issue_scatter(fdstB, rowsB)
            wait_tail(fdstB)

        plsc.subcore_barrier()
        def cp_blk(j, _):
            blk = s + j * 16
            pltpu.sync_copy(acc.at[pl.ds(blk * 128, 128)], rows)
            pltpu.sync_copy(rows, s_out.at[pl.ds(lo + blk * 128, 128)])
            if do_deg:
                pltpu.sync_copy(dacc.at[pl.ds(blk * 128, 128)], cpb1)
                pltpu.sync_copy(cpb1, d_out.at[pl.ds(lo + blk * 128, 128)])
            return 0

        lax.fori_loop(0, ncp // 16, cp_blk, 0)
        plsc.subcore_barrier()
        return 0

    lax.fori_loop(0, npass, pass_body, 0)


def _make_agg_kernel(layer0: bool):
    mesh = plsc.VectorSubcoreMesh(core_axis_name="c", subcore_axis_name="s",
                                  num_cores=2, num_subcores=16)

    out_type = [
        jax.ShapeDtypeStruct((2 * R_BIG * _RATES[2], H), jnp.float32),
        jax.ShapeDtypeStruct((2 * R_BIG * _REV[2], H), jnp.float32),
        jax.ShapeDtypeStruct((2 * R_BIG * _TO[2], H), jnp.float32),
    ]
    if layer0:
        out_type.append(jax.ShapeDtypeStruct((2 * R_TAG, H), jnp.float32))
        out_type += [
            jax.ShapeDtypeStruct((2 * R_BIG * _RATES[2],), jnp.float32),
            jax.ShapeDtypeStruct((2 * R_BIG * _REV[2],), jnp.float32),
            jax.ShapeDtypeStruct((2 * R_BIG * _TO[2],), jnp.float32),
            jax.ShapeDtypeStruct((2 * R_TAG,), jnp.float32),
        ]

    scratch = [
        pltpu.VMEM_SHARED((ACC_ROWS, H), jnp.float32),
        pltpu.VMEM_SHARED((ACC_ROWS,), jnp.float32),
        pltpu.VMEM((CH,), jnp.int32),
        pltpu.VMEM((CH,), jnp.int32),
        pltpu.VMEM((CH,), jnp.int32),
        pltpu.VMEM((CH,), jnp.int32),
        pltpu.SemaphoreType.DMA,
        pltpu.SemaphoreType.DMA,
        pltpu.VMEM((512,), jnp.int32),
        pltpu.VMEM((512,), jnp.int32),
        pltpu.VMEM((GB,), jnp.int32),
        pltpu.VMEM((GB,), jnp.int32),
        pltpu.VMEM((GB,), jnp.int32),
        pltpu.VMEM((GB,), jnp.int32),
        pltpu.VMEM((GB, H), jnp.float32),
        pltpu.VMEM((GB, H), jnp.float32),
        pltpu.VMEM((GB,), jnp.float32),
        pltpu.VMEM((GB,), jnp.float32),
        pltpu.VMEM((GB,), jnp.float32),
        pltpu.SemaphoreType.DMA,
        pltpu.SemaphoreType.DMA,
        pltpu.SemaphoreType.DMA,
        pltpu.SemaphoreType.DMA,
    ]

    def body(h_user, h_movie, h_tag,
             r_src, r_dst, v_src, v_dst, o_src, o_dst, ht_src, ht_dst,
             ones1h, zer, zer1, *rest):
        if layer0:
            (s_rates, s_rev, s_to, s_ht, d_rates, d_rev, d_to, d_ht,
             acc, dacc, src_ch, dst_ch, src_ch2, dst_ch2, semI, semI2,
             cbs, cbd, fsrcA, fdstA, fsrcB,
             fdstB, rowsA, rowsB, cpb1, z1, ones_v, semA, semB,
             semS, semD) = rest
        else:
            (s_rates, s_rev, s_to,
             acc, dacc, src_ch, dst_ch, src_ch2, dst_ch2, semI, semI2,
             cbs, cbd, fsrcA, fdstA, fsrcB,
             fdstB, rowsA, rowsB, cpb1, z1, ones_v, semA, semB,
             semS, semD) = rest
            d_rates = d_rev = d_to = d_ht = s_ht = None
        c = lax.axis_index("c")
        s = lax.axis_index("s")
        pltpu.sync_copy(ones1h, ones_v)
        pltpu.sync_copy(zer1, z1)
        common = dict(zer=zer, acc=acc, dacc=dacc, src_ch=src_ch,
                      dst_ch=dst_ch, src_ch2=src_ch2, dst_ch2=dst_ch2,
                      semI=semI, semI2=semI2, cbs=cbs, cbd=cbd, fsrcA=fsrcA,
                      fdstA=fdstA, fsrcB=fsrcB, fdstB=fdstB, rowsA=rowsA,
                      rowsB=rowsB, cpb1=cpb1, z1=z1, ones_v=ones_v,
                      semA=semA, semB=semB, semS=semS, semD=semD)
        _run_agg(c, s, h_user, r_src, r_dst, EPR, _RATES[1], _RATES[2],
                 s_rates, d_rates, **common)
        _run_agg(c, s, h_movie, v_src, v_dst, EPR, _REV[1], _REV[2],
                 s_rev, d_rev, **common)
        _run_agg(c, s, h_tag, o_src, o_dst, EPT, _TO[1], _TO[2],
                 s_to, d_to, **common)
        if layer0:
            _run_agg(c, s, h_movie, ht_src, ht_dst, EPT, _HT[1], _HT[2],
                     s_ht, d_ht, **common)

    return pl.kernel(body, out_type=tuple(out_type), mesh=mesh,
                     scratch_types=scratch,
                     compiler_params=pltpu.CompilerParams(
                         needs_layout_passes=False))


_agg_cache = {}


def _agg(layer0, *args):
    if layer0 not in _agg_cache:
        _agg_cache[layer0] = _make_agg_kernel(layer0)
    return _agg_cache[layer0](*args)


_agg_l0 = functools.partial(_agg, True)
_agg_l1 = functools.partial(_agg, False)


def _proj_movie(movie_genre_p, W_proj, b_proj):
    def body(g_ref, w_ref, b_ref, o_ref):
        o_ref[...] = jnp.dot(g_ref[...], w_ref[...],
                             preferred_element_type=jnp.float32) + b_ref[...]

    return pl.pallas_call(
        body,
        grid=(NMP // BK,),
        in_specs=[pl.BlockSpec((BK, D_FEAT), lambda i: (i, 0)),
                  pl.BlockSpec((D_FEAT, H), lambda i: (0, 0)),
                  pl.BlockSpec((1, H), lambda i: (0, 0))],
        out_specs=pl.BlockSpec((BK, H), lambda i: (i, 0)),
        out_shape=jax.ShapeDtypeStruct((NMP, H), jnp.float32),
    )(movie_genre_p, W_proj, b_proj.reshape(1, H))


def _eye128():
    ri = lax.broadcasted_iota(jnp.int32, (H, H), 0)
    ci = lax.broadcasted_iota(jnp.int32, (H, H), 1)
    return (ri == ci).astype(jnp.float32)


def _dcol(eye, d2, b):
    drow = d2[b:b + 1, :]
    return lax.dot_general(eye, drow, (((1,), (1,)), ((), ())),
                           preferred_element_type=jnp.float32)


def _dense1(h, s_n, deg2, Ws, Wn, b):
    n = h.shape[0]

    def body(h_ref, s_ref, d_ref, ws_ref, wn_ref, b_ref, o_ref):
        eye = _eye128()
        x = jnp.dot(h_ref[...], ws_ref[...],
                    preferred_element_type=jnp.float32) + b_ref[...]
        for bb in range(BK // H):
            dcol = _dcol(eye, d_ref[...], bb)
            mb = s_ref[pl.ds(bb * H, H), :] * (1.0 / jnp.maximum(dcol, 1.0))
            zb = x[bb * H:(bb + 1) * H, :] + jnp.dot(
                mb, wn_ref[...], preferred_element_type=jnp.float32)
            o_ref[pl.ds(bb * H, H), :] = jnp.where(zb >= 0, zb, 0.1 * zb)

    row = lambda i: (i, 0)
    full = lambda i: (0, 0)
    return pl.pallas_call(
        body,
        grid=(n // BK,),
        in_specs=[pl.BlockSpec((BK, H), row),
                  pl.BlockSpec((BK, H), row),
                  pl.BlockSpec((BK // H, H), row),
                  pl.BlockSpec((H, H), full),
                  pl.BlockSpec((H, H), full),
                  pl.BlockSpec((1, H), full)],
        out_specs=pl.BlockSpec((BK, H), row),
        out_shape=jax.ShapeDtypeStruct((n, H), jnp.float32),
    )(h, s_n, deg2, Ws, Wn, b.reshape(1, H))


def _dense2(h, s1, d1, s2, d2, Wsa, Wsb, Wn1, Wn2, b):
    n = h.shape[0]

    def body(h_ref, s1_ref, d1_ref, s2_ref, d2_ref,
             wsa_ref, wsb_ref, wn1_ref, wn2_ref, b_ref, o_ref):
        eye = _eye128()
        ws = wsa_ref[...] + wsb_ref[...]
        x = jnp.dot(h_ref[...], ws,
                    preferred_element_type=jnp.float32) + b_ref[...]
        for bb in range(BK // H):
            c1 = _dcol(eye, d1_ref[...], bb)
            c2 = _dcol(eye, d2_ref[...], bb)
            m1 = s1_ref[pl.ds(bb * H, H), :] * (1.0 / jnp.maximum(c1, 1.0))
            m2 = s2_ref[pl.ds(bb * H, H), :] * (1.0 / jnp.maximum(c2, 1.0))
            zb = x[bb * H:(bb + 1) * H, :]
            zb = zb + jnp.dot(m1, wn1_ref[...], preferred_element_type=jnp.float32)
            zb = zb + jnp.dot(m2, wn2_ref[...], preferred_element_type=jnp.float32)
            o_ref[pl.ds(bb * H, H), :] = jnp.where(zb >= 0, zb, 0.1 * zb)

    row = lambda i: (i, 0)
    full = lambda i: (0, 0)
    return pl.pallas_call(
        body,
        grid=(n // BK,),
        in_specs=[pl.BlockSpec((BK, H), row),
                  pl.BlockSpec((BK, H), row),
                  pl.BlockSpec((BK // H, H), row),
                  pl.BlockSpec((BK, H), row),
                  pl.BlockSpec((BK // H, H), row),
                  pl.BlockSpec((H, H), full),
                  pl.BlockSpec((H, H), full),
                  pl.BlockSpec((H, H), full),
                  pl.BlockSpec((H, H), full),
                  pl.BlockSpec((1, H), full)],
        out_specs=pl.BlockSpec((BK, H), row),
        out_shape=jax.ShapeDtypeStruct((n, H), jnp.float32),
    )(h, s1, d1, s2, d2, Wsa, Wsb, Wn1, Wn2, b.reshape(1, H))


def _pad_edges(idx, epad, fill):
    pad = jnp.full((epad - idx.shape[0],), fill, dtype=jnp.int32)
    return jnp.concatenate([idx.astype(jnp.int32), pad])


def _pad_rows(x, n):
    return jnp.pad(x, ((0, n - x.shape[0]), (0, 0)))


def _deg2(d_flat, npadded):
    return d_flat.reshape(-1, H)[: npadded // H]


@jax.jit
def kernel(movie_genre, rates_src, rates_dst, has_tag_src, has_tag_dst,
           user_emb, tag_emb, W_proj, b_proj, W_self, W_neigh, b_conv):
    r_src = _pad_edges(rates_src, EPR, 0)
    r_dst = _pad_edges(rates_dst, EPR, NM)
    v_src = _pad_edges(rates_dst, EPR, 0)
    v_dst = _pad_edges(rates_src, EPR, NU)
    ht_src = _pad_edges(has_tag_src, EPT, 0)
    ht_dst = _pad_edges(has_tag_dst, EPT, NT)
    o_src = _pad_edges(has_tag_dst, EPT, 0)
    o_dst = _pad_edges(has_tag_src, EPT, NM)

    ones1 = jnp.ones((GB,), jnp.float32)
    zer = jnp.zeros((GB, H), jnp.float32)
    zer1 = jnp.zeros((GB,), jnp.float32)

    h_u0 = _pad_rows(user_emb, NUP)
    h_t0 = _pad_rows(tag_emb, NTP)
    h_m0 = _proj_movie(_pad_rows(movie_genre, NMP), W_proj, b_proj)

    (s_rates, s_rev, s_to, s_ht, d_rates, d_rev, d_to, d_ht) = _agg_l0(
        h_u0, h_m0, h_t0,
        r_src, r_dst, v_src, v_dst, o_src, o_dst, ht_src, ht_dst,
        ones1, zer, zer1)

    d2_rates = _deg2(d_rates, NMP)
    d2_rev = _deg2(d_rev, NUP)
    d2_to = _deg2(d_to, NMP)
    d2_ht = _deg2(d_ht, NTP)
    h_u1 = _dense1(h_u0, s_rev[:NUP], d2_rev,
                   W_self[0, 1], W_neigh[0, 1], b_conv[0, 1])
    h_t1 = _dense1(h_t0, s_ht[:NTP], d2_ht,
                   W_self[0, 2], W_neigh[0, 2], b_conv[0, 2])
    h_m1 = _dense2(h_m0, s_rates[:NMP], d2_rates, s_to[:NMP], d2_to,
                   W_self[0, 0], W_self[0, 3], W_neigh[0, 0], W_neigh[0, 3],
                   b_conv[0, 0] + b_conv[0, 3])

    (s_rates1, s_rev1, s_to1) = _agg_l1(
        h_u1, h_m1, h_t1,
        r_src, r_dst, v_src, v_dst, o_src, o_dst, ht_src, ht_dst,
        ones1, zer, zer1)

    h_u2 = _dense1(h_u1, s_rev1[:NUP], d2_rev,
                   W_self[1, 1], W_neigh[1, 1], b_conv[1, 1])
    h_m2 = _dense2(h_m1, s_rates1[:NMP], d2_rates, s_to1[:NMP], d2_to,
                   W_self[1, 0], W_self[1, 3], W_neigh[1, 0], W_neigh[1, 3],
                   b_conv[1, 0] + b_conv[1, 3])
    return (h_u2[:NU], h_m2[:NM])
